# baseline (speedup 1.0000x reference)
import functools

import jax
import jax.numpy as jnp
import numpy as np
from jax import lax
from jax.experimental import pallas as pl

SEQ_LEN = 720
ENC_IN = 321
PERIOD = 12
NGROUP = SEQ_LEN // PERIOD
NGROUP_PAD = 64
TOPK = 30
NBIN = SEQ_LEN // 2 + 1
NBIN_PAD = 368
EPS = 1e-8
_INF_BITS = 0x7F800000

_BF = jnp.bfloat16
_F32 = jnp.float32


def _split(v):
    hi = v.astype(_BF)
    lo = (v - hi.astype(_F32)).astype(_BF)
    return hi, lo


def _mm(a, b):
    return jnp.dot(a, b, preferred_element_type=_F32)


def _mm_x2(ahi, alo, bhi, blo):
    return _mm(ahi, bhi) + (_mm(ahi, blo) + _mm(alo, bhi))


def _dft_body(x_ref, a_ref, e_ref, chi_ref, clo_ref, shi_ref, slo_ref,
              tcos_ref, tsin_ref, out_ref):
    x = x_ref[0]
    xhi, xlo = _split(x)
    x2 = x * x
    x2hi, x2lo = _split(x2)
    a = a_ref[...]
    mean = (_mm(a, xhi) + _mm(a, xlo)) * (1.0 / PERIOD)
    msq = (_mm(a, x2hi) + _mm(a, x2lo)) * (1.0 / PERIOD)
    var = (msq - mean * mean) * (PERIOD / (PERIOD - 1))
    rinv = 1.0 / (jnp.sqrt(jnp.maximum(var, 0.0)) + EPS)
    e = e_ref[...]
    mhi, mlo = _split(mean)
    rhi, rlo = _split(rinv)
    mb = _mm(e, mhi) + _mm(e, mlo)
    rb = _mm(e, rhi) + _mm(e, rlo)
    ni = (x - mb) * rb

    nhi, nlo = _split(ni)
    re = _mm_x2(chi_ref[...], clo_ref[...], nhi, nlo)
    s = _mm_x2(shi_ref[...], slo_ref[...], nhi, nlo)
    mag2 = re * re + s * s
    bits = lax.bitcast_convert_type(mag2, jnp.int32)

    j = lax.broadcasted_iota(jnp.int32, (NBIN_PAD, 1), 0)
    mu = jnp.where((j == 0) | (j == NBIN - 1), 1.0,
                   jnp.where(j < NBIN, 2.0, 0.0))
    mub = mu.astype(_BF)
    bits = jnp.where(j < NBIN, bits, -1)

    kb = lax.bitcast_convert_type(mag2.astype(_BF), jnp.int16)
    kb = jnp.where(j < NBIN, kb, jnp.int16(-1))

    def body16(_, carry):
        lo, hi = carry
        mid = lo + lax.shift_right_logical(hi - lo, 1)
        sel = jnp.where(kb >= mid.astype(jnp.int16), mub, _BF(0.0))
        cnt = jnp.sum(sel, axis=0, keepdims=True, dtype=_BF)
        ge = cnt.astype(_F32) >= float(TOPK)
        return jnp.where(ge, mid, lo), jnp.where(ge, hi, mid)

    maxbits = jnp.max(bits, axis=0, keepdims=True)
    max16_32 = lax.shift_right_arithmetic(maxbits, 16)
    hi16_0 = max16_32 + 2
    lo16_0 = jnp.maximum(max16_32 - (14 << 7), 0)
    lo16, _ = lax.fori_loop(0, 11, body16, (lo16_0, hi16_0))

    t32 = lax.shift_left(lo16, 16)
    base = jnp.maximum(t32 - 32769, 0)
    off = lax.shift_right_arithmetic(bits - base, 1)
    k2 = jnp.clip(off, 0, 32767).astype(jnp.int16)

    def body2(_, carry):
        lo, hi = carry
        mid = lo + lax.shift_right_logical(hi - lo, 1)
        sel = jnp.where(k2 >= mid.astype(jnp.int16), mub, _BF(0.0))
        cnt = jnp.sum(sel, axis=0, keepdims=True, dtype=_BF)
        ge = cnt.astype(_F32) >= float(TOPK)
        return jnp.where(ge, mid, lo), jnp.where(ge, hi, mid)

    z = jnp.zeros_like(t32)
    lo2, hi2 = lax.fori_loop(0, 15, body2, (z, z + 32768))
    lo = base + lax.shift_left(lo2, 1)
    hi = base + lax.shift_left(lo2 + 1, 1)

    ge_hi = bits >= hi
    muf = mu * jnp.ones((1, ENC_IN), _F32)
    cnt_hi = jnp.sum(jnp.where(ge_hi, muf, 0.0), axis=0, keepdims=True)
    tie = bits >= lo
    cnt_tie = jnp.sum(jnp.where(tie, muf, 0.0), axis=0, keepdims=True) - cnt_hi
    rem = float(TOPK) - cnt_hi
    frac = jnp.maximum(rem, 0.0) / jnp.maximum(cnt_tie, 1.0)
    w = jnp.where(ge_hi, mu, 0.0) + jnp.where(tie & ~ge_hi, mu * frac, 0.0)

    wre = (w * re).astype(_BF)
    ws = (w * s).astype(_BF)
    xf = _mm(tcos_ref[...], wre) + _mm(tsin_ref[...], ws)
    out_ref[0] = ni - xf


@functools.lru_cache(maxsize=1)
def _consts():
    n = np.arange(SEQ_LEN)
    j = np.arange(NBIN_PAD)
    ang = 2.0 * np.pi * np.outer(j, n) / SEQ_LEN
    ccos = np.cos(ang)
    csin = np.sin(ang)
    ccos[NBIN:] = 0.0
    csin[NBIN:] = 0.0
    tcos = (ccos.T / SEQ_LEN).astype(np.float32)
    tsin = (csin.T / SEQ_LEN).astype(np.float32)

    def split(m):
        m32 = m.astype(np.float32)
        hi = m32.astype(jnp.bfloat16)
        lo = (m32 - np.asarray(hi, np.float32)).astype(jnp.bfloat16)
        return jnp.asarray(hi), jnp.asarray(lo)

    chi, clo = split(ccos)
    shi, slo = split(csin)
    a = np.zeros((NGROUP_PAD, SEQ_LEN), np.float32)
    for g in range(NGROUP):
        a[g, g * PERIOD:(g + 1) * PERIOD] = 1.0
    e = np.zeros((SEQ_LEN, NGROUP_PAD), np.float32)
    e[n, n // PERIOD] = 1.0
    return (jnp.asarray(a, dtype=jnp.bfloat16),
            jnp.asarray(e, dtype=jnp.bfloat16),
            chi, clo, shi, slo,
            jnp.asarray(tcos, dtype=jnp.bfloat16),
            jnp.asarray(tsin, dtype=jnp.bfloat16))


def kernel(batch_x, Wi_m, bi_m, Wr_m, br_m, Wo_m, bo_m, Wi_s, bi_s, Wr_s,
           br_s, Wo_s, bo_s, Wf1, bf1, Wf2, bf2, weight):
    bs = batch_x.shape[0]
    a, e, chi, clo, shi, slo, tcos, tsin = _consts()
    full = lambda shape: pl.BlockSpec(shape, lambda b: (0,) * len(shape))
    out = pl.pallas_call(
        _dft_body,
        grid=(bs,),
        in_specs=[
            pl.BlockSpec((1, SEQ_LEN, ENC_IN), lambda b: (b, 0, 0)),
            full((NGROUP_PAD, SEQ_LEN)),
            full((SEQ_LEN, NGROUP_PAD)),
            full((NBIN_PAD, SEQ_LEN)),
            full((NBIN_PAD, SEQ_LEN)),
            full((NBIN_PAD, SEQ_LEN)),
            full((NBIN_PAD, SEQ_LEN)),
            full((SEQ_LEN, NBIN_PAD)),
            full((SEQ_LEN, NBIN_PAD)),
        ],
        out_specs=pl.BlockSpec((1, SEQ_LEN, ENC_IN), lambda b: (b, 0, 0)),
        out_shape=jax.ShapeDtypeStruct((bs, SEQ_LEN, ENC_IN), jnp.float32),
    )(batch_x, a, e, chi, clo, shi, slo, tcos, tsin)
    return out


# speedup vs baseline: 1.0327x; 1.0327x over previous
import functools

import jax
import jax.numpy as jnp
import numpy as np
from jax import lax
from jax.experimental import pallas as pl

SEQ_LEN = 720
ENC_IN = 321
PERIOD = 12
NGROUP = SEQ_LEN // PERIOD
NGROUP_PAD = 64
TOPK = 30
NBIN = SEQ_LEN // 2 + 1
NBIN_PAD = 368
EPS = 1e-8
_INF_BITS = 0x7F800000

_BF = jnp.bfloat16
_F32 = jnp.float32


def _split(v):
    hi = v.astype(_BF)
    lo = (v - hi.astype(_F32)).astype(_BF)
    return hi, lo


def _mm(a, b):
    return jnp.dot(a, b, preferred_element_type=_F32)


def _mm_x2(ahi, alo, bhi, blo):
    return _mm(ahi, bhi) + (_mm(ahi, blo) + _mm(alo, bhi))


def _dft_body(x_ref, a_ref, e_ref, chi_ref, clo_ref, shi_ref, slo_ref,
              tcos_ref, tsin_ref, out_ref):
    x = x_ref[0]
    xhi, xlo = _split(x)
    x2 = x * x
    x2hi, x2lo = _split(x2)
    a = a_ref[...]
    mean = (_mm(a, xhi) + _mm(a, xlo)) * (1.0 / PERIOD)
    msq = (_mm(a, x2hi) + _mm(a, x2lo)) * (1.0 / PERIOD)
    var = (msq - mean * mean) * (PERIOD / (PERIOD - 1))
    rinv = 1.0 / (jnp.sqrt(jnp.maximum(var, 0.0)) + EPS)
    e = e_ref[...]
    mhi, mlo = _split(mean)
    rhi, rlo = _split(rinv)
    mb = _mm(e, mhi) + _mm(e, mlo)
    rb = _mm(e, rhi) + _mm(e, rlo)
    ni = (x - mb) * rb

    nhi, nlo = _split(ni)
    re = _mm_x2(chi_ref[...], clo_ref[...], nhi, nlo)
    s = _mm_x2(shi_ref[...], slo_ref[...], nhi, nlo)
    mag2 = re * re + s * s
    bits = lax.bitcast_convert_type(mag2, jnp.int32)

    j = lax.broadcasted_iota(jnp.int32, (NBIN_PAD, 1), 0)
    mu = jnp.where((j == 0) | (j == NBIN - 1), 1.0,
                   jnp.where(j < NBIN, 2.0, 0.0))
    mub = mu.astype(_BF)
    bits = jnp.where(j < NBIN, bits, -1)

    kb = lax.bitcast_convert_type(mag2.astype(_BF), jnp.int16)
    kb = jnp.where(j < NBIN, kb, jnp.int16(-1))

    def body16(_, carry):
        lo, hi = carry
        mid = lo + lax.shift_right_logical(hi - lo, 1)
        sel = jnp.where(kb >= mid.astype(jnp.int16), mub, _BF(0.0))
        cnt = jnp.sum(sel, axis=0, keepdims=True, dtype=_BF)
        ge = cnt >= _BF(float(TOPK))
        return jnp.where(ge, mid, lo), jnp.where(ge, hi, mid)

    maxbits = jnp.max(bits, axis=0, keepdims=True)
    max16_32 = lax.shift_right_arithmetic(maxbits, 16)
    hi16_0 = max16_32 + 2
    lo16_0 = jnp.maximum(max16_32 - (14 << 7), 0)
    lo16, _ = lax.fori_loop(0, 11, body16, (lo16_0, hi16_0))

    t32 = lax.shift_left(lo16, 16)
    lo0 = jnp.maximum(t32 - 32769, 0)
    hi0 = t32 + 32769

    def body(_, carry):
        lo, hi = carry
        mid = lo + lax.shift_right_logical(hi - lo, 1)
        cnt = jnp.sum(jnp.where(bits >= mid, mu, 0.0), axis=0, keepdims=True)
        ge = cnt >= float(TOPK)
        return jnp.where(ge, mid, lo), jnp.where(ge, hi, mid)

    lo, hi = lax.fori_loop(0, 11, body, (lo0, hi0))

    ge_hi = bits >= hi
    muf = mu * jnp.ones((1, ENC_IN), _F32)
    cnt_hi = jnp.sum(jnp.where(ge_hi, muf, 0.0), axis=0, keepdims=True)
    tie = bits >= lo
    cnt_tie = jnp.sum(jnp.where(tie, muf, 0.0), axis=0, keepdims=True) - cnt_hi
    rem = float(TOPK) - cnt_hi
    frac = jnp.maximum(rem, 0.0) / jnp.maximum(cnt_tie, 1.0)
    w = jnp.where(ge_hi, mu, 0.0) + jnp.where(tie & ~ge_hi, mu * frac, 0.0)

    wre = (w * re).astype(_BF)
    ws = (w * s).astype(_BF)
    xf = _mm(tcos_ref[...], wre) + _mm(tsin_ref[...], ws)
    out_ref[0] = ni - xf


@functools.lru_cache(maxsize=1)
def _consts():
    n = np.arange(SEQ_LEN)
    j = np.arange(NBIN_PAD)
    ang = 2.0 * np.pi * np.outer(j, n) / SEQ_LEN
    ccos = np.cos(ang)
    csin = np.sin(ang)
    ccos[NBIN:] = 0.0
    csin[NBIN:] = 0.0
    tcos = (ccos.T / SEQ_LEN).astype(np.float32)
    tsin = (csin.T / SEQ_LEN).astype(np.float32)

    def split(m):
        m32 = m.astype(np.float32)
        hi = m32.astype(jnp.bfloat16)
        lo = (m32 - np.asarray(hi, np.float32)).astype(jnp.bfloat16)
        return jnp.asarray(hi), jnp.asarray(lo)

    chi, clo = split(ccos)
    shi, slo = split(csin)
    a = np.zeros((NGROUP_PAD, SEQ_LEN), np.float32)
    for g in range(NGROUP):
        a[g, g * PERIOD:(g + 1) * PERIOD] = 1.0
    e = np.zeros((SEQ_LEN, NGROUP_PAD), np.float32)
    e[n, n // PERIOD] = 1.0
    return (jnp.asarray(a, dtype=jnp.bfloat16),
            jnp.asarray(e, dtype=jnp.bfloat16),
            chi, clo, shi, slo,
            jnp.asarray(tcos, dtype=jnp.bfloat16),
            jnp.asarray(tsin, dtype=jnp.bfloat16))


def kernel(batch_x, Wi_m, bi_m, Wr_m, br_m, Wo_m, bo_m, Wi_s, bi_s, Wr_s,
           br_s, Wo_s, bo_s, Wf1, bf1, Wf2, bf2, weight):
    bs = batch_x.shape[0]
    a, e, chi, clo, shi, slo, tcos, tsin = _consts()
    full = lambda shape: pl.BlockSpec(shape, lambda b: (0,) * len(shape))
    out = pl.pallas_call(
        _dft_body,
        grid=(bs,),
        in_specs=[
            pl.BlockSpec((1, SEQ_LEN, ENC_IN), lambda b: (b, 0, 0)),
            full((NGROUP_PAD, SEQ_LEN)),
            full((SEQ_LEN, NGROUP_PAD)),
            full((NBIN_PAD, SEQ_LEN)),
            full((NBIN_PAD, SEQ_LEN)),
            full((NBIN_PAD, SEQ_LEN)),
            full((NBIN_PAD, SEQ_LEN)),
            full((SEQ_LEN, NBIN_PAD)),
            full((SEQ_LEN, NBIN_PAD)),
        ],
        out_specs=pl.BlockSpec((1, SEQ_LEN, ENC_IN), lambda b: (b, 0, 0)),
        out_shape=jax.ShapeDtypeStruct((bs, SEQ_LEN, ENC_IN), jnp.float32),
    )(batch_x, a, e, chi, clo, shi, slo, tcos, tsin)
    return out


# speedup vs baseline: 1.0454x; 1.0123x over previous
import functools

import jax
import jax.numpy as jnp
import numpy as np
from jax import lax
from jax.experimental import pallas as pl

SEQ_LEN = 720
ENC_IN = 321
PERIOD = 12
NGROUP = SEQ_LEN // PERIOD
NGROUP_PAD = 64
TOPK = 30
NBIN = SEQ_LEN // 2 + 1
NBIN_PAD = 368
EPS = 1e-8
_INF_BITS = 0x7F800000

_BF = jnp.bfloat16
_F32 = jnp.float32


def _split(v):
    hi = v.astype(_BF)
    lo = (v - hi.astype(_F32)).astype(_BF)
    return hi, lo


def _mm(a, b):
    return jnp.dot(a, b, preferred_element_type=_F32)


def _mm_x2(ahi, alo, bhi, blo):
    return _mm(ahi, bhi) + (_mm(ahi, blo) + _mm(alo, bhi))


def _dft_body(x_ref, a_ref, e_ref, chi_ref, clo_ref, shi_ref, slo_ref,
              tcos_ref, tsin_ref, out_ref):
    for _i in range(x_ref.shape[0]):
        _one_batch(x_ref, _i, a_ref, e_ref, chi_ref, clo_ref, shi_ref,
                   slo_ref, tcos_ref, tsin_ref, out_ref)


def _one_batch(x_ref, _i, a_ref, e_ref, chi_ref, clo_ref, shi_ref, slo_ref,
               tcos_ref, tsin_ref, out_ref):
    x = x_ref[_i]
    xhi, xlo = _split(x)
    x2 = x * x
    x2hi, x2lo = _split(x2)
    a = a_ref[...]
    mean = (_mm(a, xhi) + _mm(a, xlo)) * (1.0 / PERIOD)
    msq = (_mm(a, x2hi) + _mm(a, x2lo)) * (1.0 / PERIOD)
    var = (msq - mean * mean) * (PERIOD / (PERIOD - 1))
    rinv = 1.0 / (jnp.sqrt(jnp.maximum(var, 0.0)) + EPS)
    e = e_ref[...]
    mhi, mlo = _split(mean)
    rhi, rlo = _split(rinv)
    mb = _mm(e, mhi) + _mm(e, mlo)
    rb = _mm(e, rhi) + _mm(e, rlo)
    ni = (x - mb) * rb

    nhi, nlo = _split(ni)
    re = _mm_x2(chi_ref[...], clo_ref[...], nhi, nlo)
    s = _mm_x2(shi_ref[...], slo_ref[...], nhi, nlo)
    mag2 = re * re + s * s
    bits = lax.bitcast_convert_type(mag2, jnp.int32)

    j = lax.broadcasted_iota(jnp.int32, (NBIN_PAD, 1), 0)
    mu = jnp.where((j == 0) | (j == NBIN - 1), 1.0,
                   jnp.where(j < NBIN, 2.0, 0.0))
    mub = mu.astype(_BF)
    bits = jnp.where(j < NBIN, bits, -1)

    kb = lax.bitcast_convert_type(mag2.astype(_BF), jnp.int16)
    kb = jnp.where(j < NBIN, kb, jnp.int16(-1))

    def body16(_, carry):
        lo, hi = carry
        mid = lo + lax.shift_right_logical(hi - lo, 1)
        sel = jnp.where(kb >= mid.astype(jnp.int16), mub, _BF(0.0))
        cnt = jnp.sum(sel, axis=0, keepdims=True, dtype=_BF)
        ge = cnt >= _BF(float(TOPK))
        return jnp.where(ge, mid, lo), jnp.where(ge, hi, mid)

    maxbits = jnp.max(bits, axis=0, keepdims=True)
    max16_32 = lax.shift_right_arithmetic(maxbits, 16)
    hi16_0 = max16_32 + 2
    lo16_0 = jnp.maximum(max16_32 - (14 << 7), 0)
    lo16, _ = lax.fori_loop(0, 11, body16, (lo16_0, hi16_0))

    t32 = lax.shift_left(lo16, 16)
    lo0 = jnp.maximum(t32 - 32769, 0)
    hi0 = t32 + 32769

    def body(_, carry):
        lo, hi = carry
        mid = lo + lax.shift_right_logical(hi - lo, 1)
        cnt = jnp.sum(jnp.where(bits >= mid, mu, 0.0), axis=0, keepdims=True)
        ge = cnt >= float(TOPK)
        return jnp.where(ge, mid, lo), jnp.where(ge, hi, mid)

    lo, hi = lax.fori_loop(0, 11, body, (lo0, hi0))

    ge_hi = bits >= hi
    muf = mu * jnp.ones((1, ENC_IN), _F32)
    cnt_hi = jnp.sum(jnp.where(ge_hi, muf, 0.0), axis=0, keepdims=True)
    tie = bits >= lo
    cnt_tie = jnp.sum(jnp.where(tie, muf, 0.0), axis=0, keepdims=True) - cnt_hi
    rem = float(TOPK) - cnt_hi
    frac = jnp.maximum(rem, 0.0) / jnp.maximum(cnt_tie, 1.0)
    w = jnp.where(ge_hi, mu, 0.0) + jnp.where(tie & ~ge_hi, mu * frac, 0.0)

    wre = (w * re).astype(_BF)
    ws = (w * s).astype(_BF)
    xf = _mm(tcos_ref[...], wre) + _mm(tsin_ref[...], ws)
    out_ref[_i] = ni - xf


@functools.lru_cache(maxsize=1)
def _consts():
    n = np.arange(SEQ_LEN)
    j = np.arange(NBIN_PAD)
    ang = 2.0 * np.pi * np.outer(j, n) / SEQ_LEN
    ccos = np.cos(ang)
    csin = np.sin(ang)
    ccos[NBIN:] = 0.0
    csin[NBIN:] = 0.0
    tcos = (ccos.T / SEQ_LEN).astype(np.float32)
    tsin = (csin.T / SEQ_LEN).astype(np.float32)

    def split(m):
        m32 = m.astype(np.float32)
        hi = m32.astype(jnp.bfloat16)
        lo = (m32 - np.asarray(hi, np.float32)).astype(jnp.bfloat16)
        return jnp.asarray(hi), jnp.asarray(lo)

    chi, clo = split(ccos)
    shi, slo = split(csin)
    a = np.zeros((NGROUP_PAD, SEQ_LEN), np.float32)
    for g in range(NGROUP):
        a[g, g * PERIOD:(g + 1) * PERIOD] = 1.0
    e = np.zeros((SEQ_LEN, NGROUP_PAD), np.float32)
    e[n, n // PERIOD] = 1.0
    return (jnp.asarray(a, dtype=jnp.bfloat16),
            jnp.asarray(e, dtype=jnp.bfloat16),
            chi, clo, shi, slo,
            jnp.asarray(tcos, dtype=jnp.bfloat16),
            jnp.asarray(tsin, dtype=jnp.bfloat16))


def kernel(batch_x, Wi_m, bi_m, Wr_m, br_m, Wo_m, bo_m, Wi_s, bi_s, Wr_s,
           br_s, Wo_s, bo_s, Wf1, bf1, Wf2, bf2, weight):
    bs = batch_x.shape[0]
    a, e, chi, clo, shi, slo, tcos, tsin = _consts()
    full = lambda shape: pl.BlockSpec(shape, lambda b: (0,) * len(shape))
    out = pl.pallas_call(
        _dft_body,
        grid=(bs // 2,),
        in_specs=[
            pl.BlockSpec((2, SEQ_LEN, ENC_IN), lambda b: (b, 0, 0)),
            full((NGROUP_PAD, SEQ_LEN)),
            full((SEQ_LEN, NGROUP_PAD)),
            full((NBIN_PAD, SEQ_LEN)),
            full((NBIN_PAD, SEQ_LEN)),
            full((NBIN_PAD, SEQ_LEN)),
            full((NBIN_PAD, SEQ_LEN)),
            full((SEQ_LEN, NBIN_PAD)),
            full((SEQ_LEN, NBIN_PAD)),
        ],
        out_specs=pl.BlockSpec((2, SEQ_LEN, ENC_IN), lambda b: (b, 0, 0)),
        out_shape=jax.ShapeDtypeStruct((bs, SEQ_LEN, ENC_IN), jnp.float32),
    )(batch_x, a, e, chi, clo, shi, slo, tcos, tsin)
    return out


# speedup vs baseline: 1.0515x; 1.0059x over previous
import functools

import jax
import jax.numpy as jnp
import numpy as np
from jax import lax
from jax.experimental import pallas as pl

SEQ_LEN = 720
ENC_IN = 321
PERIOD = 12
NGROUP = SEQ_LEN // PERIOD
NGROUP_PAD = 64
TOPK = 30
NBIN = SEQ_LEN // 2 + 1
NBIN_PAD = 368
EPS = 1e-8
_INF_BITS = 0x7F800000

_BF = jnp.bfloat16
_F32 = jnp.float32


def _split(v):
    hi = v.astype(_BF)
    lo = (v - hi.astype(_F32)).astype(_BF)
    return hi, lo


def _mm(a, b):
    return jnp.dot(a, b, preferred_element_type=_F32)


def _mm_x2(ahi, alo, bhi, blo):
    return _mm(ahi, bhi) + (_mm(ahi, blo) + _mm(alo, bhi))


def _dft_body(x_ref, a_ref, e_ref, chi_ref, clo_ref, shi_ref, slo_ref,
              tcos_ref, tsin_ref, out_ref):
    for _i in range(x_ref.shape[0]):
        _one_batch(x_ref, _i, a_ref, e_ref, chi_ref, clo_ref, shi_ref,
                   slo_ref, tcos_ref, tsin_ref, out_ref)


def _one_batch(x_ref, _i, a_ref, e_ref, chi_ref, clo_ref, shi_ref, slo_ref,
               tcos_ref, tsin_ref, out_ref):
    x = x_ref[_i]
    xhi, xlo = _split(x)
    x2 = x * x
    x2hi, x2lo = _split(x2)
    a = a_ref[...]
    mean = (_mm(a, xhi) + _mm(a, xlo)) * (1.0 / PERIOD)
    msq = (_mm(a, x2hi) + _mm(a, x2lo)) * (1.0 / PERIOD)
    var = (msq - mean * mean) * (PERIOD / (PERIOD - 1))
    rinv = 1.0 / (jnp.sqrt(jnp.maximum(var, 0.0)) + EPS)
    e = e_ref[...]
    mhi, mlo = _split(mean)
    rhi, rlo = _split(rinv)
    mb = _mm(e, mhi) + _mm(e, mlo)
    rb = _mm(e, rhi) + _mm(e, rlo)
    ni = (x - mb) * rb

    nhi, nlo = _split(ni)
    re = _mm_x2(chi_ref[...], clo_ref[...], nhi, nlo)
    s = _mm_x2(shi_ref[...], slo_ref[...], nhi, nlo)
    mag2 = re * re + s * s
    bits = lax.bitcast_convert_type(mag2, jnp.int32)

    j = lax.broadcasted_iota(jnp.int32, (NBIN_PAD, 1), 0)
    mu = jnp.where((j == 0) | (j == NBIN - 1), 1.0,
                   jnp.where(j < NBIN, 2.0, 0.0))
    mub = mu.astype(_BF)
    bits = jnp.where(j < NBIN, bits, -1)

    kb = lax.bitcast_convert_type(mag2.astype(_BF), jnp.int16)
    kb = jnp.where(j < NBIN, kb, jnp.int16(-1))

    def body16(_, carry):
        lo, hi = carry
        mid = lo + lax.shift_right_logical(hi - lo, 1)
        sel = jnp.where(kb >= mid.astype(jnp.int16), mub, _BF(0.0))
        cnt = jnp.sum(sel, axis=0, keepdims=True, dtype=_BF)
        ge = cnt >= _BF(float(TOPK))
        return jnp.where(ge, mid, lo), jnp.where(ge, hi, mid)

    maxbits = jnp.max(bits, axis=0, keepdims=True)
    max16_32 = lax.shift_right_arithmetic(maxbits, 16)
    hi16_0 = max16_32 + 2
    lo16_0 = jnp.maximum(max16_32 - (14 << 7), 0)
    lo16, _ = lax.fori_loop(0, 11, body16, (lo16_0, hi16_0))

    t32 = lax.shift_left(lo16, 16)
    lo0 = jnp.maximum(t32 - 32769, 0)
    hi0 = t32 + 32769

    def body(_, carry):
        lo, hi = carry
        mid = lo + lax.shift_right_logical(hi - lo, 1)
        cnt = jnp.sum(jnp.where(bits >= mid, mu, 0.0), axis=0, keepdims=True)
        ge = cnt >= float(TOPK)
        return jnp.where(ge, mid, lo), jnp.where(ge, hi, mid)

    lo, hi = lax.fori_loop(0, 11, body, (lo0, hi0))

    ge_hi = bits >= hi
    muf = mu * jnp.ones((1, ENC_IN), _F32)
    cnt_hi = jnp.sum(jnp.where(ge_hi, muf, 0.0), axis=0, keepdims=True)
    tie = bits >= lo
    cnt_tie = jnp.sum(jnp.where(tie, muf, 0.0), axis=0, keepdims=True) - cnt_hi
    rem = float(TOPK) - cnt_hi
    frac = jnp.maximum(rem, 0.0) / jnp.maximum(cnt_tie, 1.0)
    w = jnp.where(ge_hi, mu, 0.0) + jnp.where(tie & ~ge_hi, mu * frac, 0.0)

    wre = (w * re).astype(_BF)
    ws = (w * s).astype(_BF)
    xf = _mm(tcos_ref[...], wre) + _mm(tsin_ref[...], ws)
    out_ref[_i] = ni - xf


@functools.lru_cache(maxsize=1)
def _consts():
    n = np.arange(SEQ_LEN)
    j = np.arange(NBIN_PAD)
    ang = 2.0 * np.pi * np.outer(j, n) / SEQ_LEN
    ccos = np.cos(ang)
    csin = np.sin(ang)
    ccos[NBIN:] = 0.0
    csin[NBIN:] = 0.0
    tcos = (ccos.T / SEQ_LEN).astype(np.float32)
    tsin = (csin.T / SEQ_LEN).astype(np.float32)

    def split(m):
        m32 = m.astype(np.float32)
        hi = m32.astype(jnp.bfloat16)
        lo = (m32 - np.asarray(hi, np.float32)).astype(jnp.bfloat16)
        return jnp.asarray(hi), jnp.asarray(lo)

    chi, clo = split(ccos)
    shi, slo = split(csin)
    a = np.zeros((NGROUP_PAD, SEQ_LEN), np.float32)
    for g in range(NGROUP):
        a[g, g * PERIOD:(g + 1) * PERIOD] = 1.0
    e = np.zeros((SEQ_LEN, NGROUP_PAD), np.float32)
    e[n, n // PERIOD] = 1.0
    return (jnp.asarray(a, dtype=jnp.bfloat16),
            jnp.asarray(e, dtype=jnp.bfloat16),
            chi, clo, shi, slo,
            jnp.asarray(tcos, dtype=jnp.bfloat16),
            jnp.asarray(tsin, dtype=jnp.bfloat16))


def kernel(batch_x, Wi_m, bi_m, Wr_m, br_m, Wo_m, bo_m, Wi_s, bi_s, Wr_s,
           br_s, Wo_s, bo_s, Wf1, bf1, Wf2, bf2, weight):
    bs = batch_x.shape[0]
    a, e, chi, clo, shi, slo, tcos, tsin = _consts()
    full = lambda shape: pl.BlockSpec(shape, lambda b: (0,) * len(shape))
    out = pl.pallas_call(
        _dft_body,
        grid=(bs // 4,),
        in_specs=[
            pl.BlockSpec((4, SEQ_LEN, ENC_IN), lambda b: (b, 0, 0)),
            full((NGROUP_PAD, SEQ_LEN)),
            full((SEQ_LEN, NGROUP_PAD)),
            full((NBIN_PAD, SEQ_LEN)),
            full((NBIN_PAD, SEQ_LEN)),
            full((NBIN_PAD, SEQ_LEN)),
            full((NBIN_PAD, SEQ_LEN)),
            full((SEQ_LEN, NBIN_PAD)),
            full((SEQ_LEN, NBIN_PAD)),
        ],
        out_specs=pl.BlockSpec((4, SEQ_LEN, ENC_IN), lambda b: (b, 0, 0)),
        out_shape=jax.ShapeDtypeStruct((bs, SEQ_LEN, ENC_IN), jnp.float32),
    )(batch_x, a, e, chi, clo, shi, slo, tcos, tsin)
    return out
